# baseline (speedup 1.0000x reference)
import functools

import jax
import jax.numpy as jnp
from jax.experimental import pallas as pl
from jax.experimental.pallas import tpu as pltpu

_NEG = -1e30


def _lrelu(x):
    return jnp.where(x > 0, x, 0.2 * x)


def _pass1_kernel(M, R, T,
                  x_ref, x0_ref, wg1_ref, as1_ref, ad1_ref, bg1_ref,
                  wg2_ref, as2_ref, ad2_ref,
                  xw2_ref, xw20_ref, scal_ref,
                  m1_ref, s1_ref, acc1_ref, mals2_ref):
    i = pl.program_id(0)

    @pl.when(i == 0)
    def _init():
        for h in range(4):
            m1_ref[0, h] = _NEG
            s1_ref[0, h] = 0.0
        acc1_ref[...] = jnp.zeros((1, 512), jnp.float32)
        mals2_ref[0, 0] = _NEG

    base = i * R
    rid = jax.lax.broadcasted_iota(jnp.int32, (R, 1), 0)
    valid1 = (base + rid) < M

    Xt = jnp.where(valid1, x_ref[...], jnp.bfloat16(0))
    XW = jnp.dot(Xt, wg1_ref[...], preferred_element_type=jnp.float32)
    xw0 = jnp.dot(x0_ref[...], wg1_ref[...], preferred_element_type=jnp.float32)

    XW2 = jnp.zeros((R, 128), jnp.float32)
    for h in range(4):
        blk = slice(128 * h, 128 * (h + 1))
        XWh = XW[:, blk]
        xw0h = xw0[:, blk]
        ash = as1_ref[h:h + 1, :]
        adh = ad1_ref[h:h + 1, :]
        ALSh = jnp.sum(XWh * ash, axis=1, keepdims=True)
        ALDh = jnp.sum(XWh * adh, axis=1, keepdims=True)
        als0h = jnp.sum(xw0h * ash)
        ald0h = jnp.sum(xw0h * adh)
        e0j = _lrelu(als0h + ALDh)
        ejj = _lrelu(ALSh + ALDh)
        mloc = jnp.maximum(e0j, ejj)
        w0 = jnp.exp(e0j - mloc)
        wj = jnp.exp(ejj - mloc)
        den = w0 + wj
        X1h = (w0 / den) * xw0h + (wj / den) * XWh
        X1h = jnp.maximum(X1h + bg1_ref[:, blk], 0.0)
        XW2 = XW2 + jnp.dot(X1h.astype(jnp.bfloat16), wg2_ref[blk, :],
                            preferred_element_type=jnp.float32)
        ej0 = jnp.where(valid1, _lrelu(ALSh + ald0h), _NEG)
        mold = m1_ref[0, h]
        mnew = jnp.maximum(mold, jnp.max(ej0))
        scale = jnp.exp(mold - mnew)
        w = jnp.exp(ej0 - mnew)
        s1_ref[0, h] = s1_ref[0, h] * scale + jnp.sum(w)
        acc1_ref[:, blk] = acc1_ref[:, blk] * scale + jnp.sum(
            w * XWh, axis=0, keepdims=True)
        m1_ref[0, h] = mnew

    XW2 = jnp.where(valid1, XW2, 0.0)
    xw2_ref[...] = XW2.astype(jnp.bfloat16)
    als2 = jnp.sum(XW2 * as2_ref[...], axis=1, keepdims=True)
    als2 = jnp.where(valid1, als2, _NEG)
    mals2_ref[0, 0] = jnp.maximum(mals2_ref[0, 0], jnp.max(als2))

    @pl.when(i == T - 1)
    def _epilogue():
        xw2_0 = jnp.zeros((1, 128), jnp.float32)
        for h in range(4):
            blk = slice(128 * h, 128 * (h + 1))
            xw0h = xw0[:, blk]
            als0h = jnp.sum(xw0h * as1_ref[h:h + 1, :])
            ald0h = jnp.sum(xw0h * ad1_ref[h:h + 1, :])
            e00 = _lrelu(als0h + ald0h)
            mo = m1_ref[0, h]
            mf = jnp.maximum(mo, e00)
            sc_o = jnp.exp(mo - mf)
            sc_s = jnp.exp(e00 - mf)
            s = s1_ref[0, h] * sc_o + sc_s
            acc = acc1_ref[:, blk] * sc_o + sc_s * xw0h
            x1_0h = jnp.maximum(acc / s + bg1_ref[:, blk], 0.0)
            xw2_0 = xw2_0 + jnp.dot(x1_0h.astype(jnp.bfloat16), wg2_ref[blk, :],
                                    preferred_element_type=jnp.float32)
        xw20_ref[...] = xw2_0
        als2_0 = jnp.sum(xw2_0 * as2_ref[...])
        ald2_0 = jnp.sum(xw2_0 * ad2_ref[...])
        gmax = jnp.maximum(mals2_ref[0, 0], als2_0)
        lane = jax.lax.broadcasted_iota(jnp.int32, (1, 128), 1)
        scal_ref[...] = (jnp.where(lane == 0, als2_0, 0.0)
                         + jnp.where(lane == 1, ald2_0, 0.0)
                         + jnp.where(lane == 2, gmax, 0.0))


def _pass2_kernel(M, R, T,
                  xw2_ref, xw20_ref, scal_ref, as2_ref, bg2_ref,
                  qe_ref, wq1_ref, bq1_ref, wq2_ref, bq2_ref,
                  pe_ref, pr_ref, wep_ref, bep_ref,
                  wihe_ref, wihr_ref, bih_ref, whh_ref, bhh_ref,
                  wqp_ref, bqp_ref, wpp_ref, bpp_ref, wef_ref, bef_ref,
                  wp1a_ref, wp1b_ref, wp1c_ref, bp1_ref,
                  wp2_ref, bp2_ref, wp3_ref, bp3_ref, vr_ref,
                  probs_ref, vlog_ref,
                  s2_ref, acc2_ref):
    i = pl.program_id(0)

    @pl.when(i == 0)
    def _init():
        s2_ref[...] = jnp.zeros((1, 128), jnp.float32)
        acc2_ref[...] = jnp.zeros((1, 128), jnp.float32)

    lane = jax.lax.broadcasted_iota(jnp.int32, (1, 128), 1)
    scal = scal_ref[...]
    als2_0 = jnp.sum(jnp.where(lane == 0, scal, 0.0))
    ald2_0 = jnp.sum(jnp.where(lane == 1, scal, 0.0))
    gmax = jnp.sum(jnp.where(lane == 2, scal, 0.0))
    m2 = _lrelu(gmax + ald2_0)

    XW2 = xw2_ref[...].astype(jnp.float32)
    als2 = jnp.sum(XW2 * as2_ref[...], axis=1, keepdims=True)
    rid1 = jax.lax.broadcasted_iota(jnp.int32, (R, 1), 0)
    e2 = jnp.where((i * R + rid1) < M, _lrelu(als2 + ald2_0), _NEG)
    w = jnp.exp(e2 - m2)
    acc2_ref[...] = acc2_ref[...] + jnp.sum(w * XW2, axis=0, keepdims=True)
    s2_ref[...] = s2_ref[...] + jnp.sum(w)

    @pl.when(i == T - 1)
    def _epilogue():
        e00 = _lrelu(als2_0 + ald2_0)
        w00 = jnp.exp(e00 - m2)
        acc = acc2_ref[...] + w00 * xw20_ref[...]
        s = s2_ref[...] + w00
        ent = jnp.maximum(acc / s + bg2_ref[...], 0.0)

        q = jnp.maximum(jnp.dot(qe_ref[...], wq1_ref[...],
                                preferred_element_type=jnp.float32) + bq1_ref[...], 0.0)
        q = jnp.dot(q, wq2_ref[...], preferred_element_type=jnp.float32) + bq2_ref[...]

        ents = jnp.dot(pe_ref[...], wep_ref[...],
                       preferred_element_type=jnp.float32) + bep_ref[...]
        prel = pr_ref[...]
        h = jnp.zeros((1, 128), jnp.float32)
        c = jnp.zeros((1, 128), jnp.float32)
        for t in range(5):
            g = (jnp.dot(ents[t:t + 1, :], wihe_ref[...], preferred_element_type=jnp.float32)
                 + jnp.dot(prel[t:t + 1, :], wihr_ref[...], preferred_element_type=jnp.float32)
                 + bih_ref[...]
                 + jnp.dot(h, whh_ref[...], preferred_element_type=jnp.float32)
                 + bhh_ref[...])
            ig = jax.nn.sigmoid(g[:, 0:128])
            fg = jax.nn.sigmoid(g[:, 128:256])
            gg = jnp.tanh(g[:, 256:384])
            og = jax.nn.sigmoid(g[:, 384:512])
            c = fg * c + ig * gg
            h = og * jnp.tanh(c)

        qp = jnp.dot(q, wqp_ref[...], preferred_element_type=jnp.float32) + bqp_ref[...]
        pp = jnp.dot(h, wpp_ref[...], preferred_element_type=jnp.float32) + bpp_ref[...]
        ep = jnp.dot(ent, wef_ref[...], preferred_element_type=jnp.float32) + bef_ref[...]
        hh = jnp.maximum(jnp.dot(qp, wp1a_ref[...], preferred_element_type=jnp.float32)
                         + jnp.dot(pp, wp1b_ref[...], preferred_element_type=jnp.float32)
                         + jnp.dot(ep, wp1c_ref[...], preferred_element_type=jnp.float32)
                         + bp1_ref[...], 0.0)
        h2 = jnp.maximum(jnp.dot(hh, wp2_ref[...], preferred_element_type=jnp.float32)
                         + bp2_ref[...], 0.0)
        logits = jnp.dot(h2, wp3_ref[...], preferred_element_type=jnp.float32) + bp3_ref[...]

        vr = vr_ref[...]
        oh = (jax.lax.broadcasted_iota(jnp.int32, (1024, 64), 0) == vr).astype(jnp.float32)
        vl = jnp.dot(logits, oh, preferred_element_type=jnp.float32)
        mx = jnp.max(vl)
        ex = jnp.exp(vl - mx)
        probs_ref[...] = ex / jnp.sum(ex)
        vlog_ref[...] = vl


def kernel(question_emb, current_entity_emb, path_entities, path_relations,
           neighbor_entities, valid_relations,
           Wq1, bq1, Wq2, bq2, Wep, bep, W_ih, W_hh, b_ih, b_hh,
           Wg1, as1, ad1, bg1, Wg2, as2, ad2, bg2,
           Wqp, bqp, Wpp, bpp, Wef, bef, Wp1, bp1, Wp2, bp2, Wp3, bp3):
    f32 = jnp.float32
    bf16 = jnp.bfloat16
    M = neighbor_entities.shape[0]
    R = 1024
    T = (M + R - 1) // R
    NREL = Wp3.shape[1]
    NRELP = ((NREL + 127) // 128) * 128
    NV = valid_relations.shape[0]

    x0r = current_entity_emb.reshape(1, 128).astype(bf16)
    row = lambda v: v.reshape(1, -1)

    full = lambda shp: pl.BlockSpec(shp, lambda i: tuple(0 for _ in shp))
    p1 = pl.pallas_call(
        functools.partial(_pass1_kernel, M, R, T),
        grid=(T,),
        in_specs=[
            pl.BlockSpec((R, 128), lambda i: (i, 0)),
            full((1, 128)),
            full((128, 512)),
            full((4, 128)),
            full((4, 128)),
            full((1, 512)),
            full((512, 128)),
            full((1, 128)),
            full((1, 128)),
        ],
        out_specs=[
            pl.BlockSpec((R, 128), lambda i: (i, 0)),
            full((1, 128)),
            full((1, 128)),
        ],
        out_shape=[
            jax.ShapeDtypeStruct((T * R, 128), bf16),
            jax.ShapeDtypeStruct((1, 128), f32),
            jax.ShapeDtypeStruct((1, 128), f32),
        ],
        scratch_shapes=[
            pltpu.SMEM((1, 4), f32),
            pltpu.SMEM((1, 4), f32),
            pltpu.VMEM((1, 512), f32),
            pltpu.SMEM((1, 1), f32),
        ],
    )
    xw2_buf, xw2_0, scal = p1(
        neighbor_entities.astype(bf16), x0r, Wg1.astype(bf16), as1, ad1,
        row(bg1), Wg2.astype(bf16), as2, ad2)

    wp3p = jnp.zeros((Wp3.shape[0], NRELP), f32).at[:, :NREL].set(Wp3)
    bp3p = jnp.zeros((1, NRELP), f32).at[0, :NREL].set(bp3)

    p2 = pl.pallas_call(
        functools.partial(_pass2_kernel, M, R, T),
        grid=(T,),
        in_specs=[
            pl.BlockSpec((R, 128), lambda i: (i, 0)),
            full((1, 128)),
            full((1, 128)),
            full((1, 128)),
            full((1, 128)),
            full((1, 128)),
            full((128, 128)), full((1, 128)),
            full((128, 128)), full((1, 128)),
            full((5, 128)),
            full((5, 128)),
            full((128, 128)), full((1, 128)),
            full((128, 512)),
            full((128, 512)),
            full((1, 512)),
            full((128, 512)),
            full((1, 512)),
            full((128, 128)), full((1, 128)),
            full((128, 128)), full((1, 128)),
            full((128, 128)), full((1, 128)),
            full((128, 128)), full((128, 128)), full((128, 128)),
            full((1, 128)),
            full((128, 64)), full((1, 64)),
            full((64, NRELP)), full((1, NRELP)),
            full((1, NV)),
        ],
        out_specs=[full((1, NV)), full((1, NV))],
        out_shape=[
            jax.ShapeDtypeStruct((1, NV), f32),
            jax.ShapeDtypeStruct((1, NV), f32),
        ],
        scratch_shapes=[
            pltpu.VMEM((1, 128), f32),
            pltpu.VMEM((1, 128), f32),
        ],
    )
    probs, vlog = p2(
        xw2_buf, xw2_0, scal, as2, row(bg2),
        row(question_emb), Wq1, row(bq1), Wq2, row(bq2),
        path_entities[:-1], path_relations, Wep, row(bep),
        W_ih[:, :128].T, W_ih[:, 128:].T, row(b_ih), W_hh.T, row(b_hh),
        Wqp, row(bqp), Wpp, row(bpp), Wef, row(bef),
        Wp1[0:128], Wp1[128:256], Wp1[256:384], row(bp1),
        Wp2, row(bp2), wp3p, bp3p, valid_relations.reshape(1, -1),
    )
    return probs.reshape(-1), vlog.reshape(-1)


# speedup vs baseline: 1.1409x; 1.1409x over previous
import functools

import jax
import jax.numpy as jnp
from jax.experimental import pallas as pl
from jax.experimental.pallas import tpu as pltpu

_NEG = -1e30


def _lrelu(x):
    return jnp.where(x > 0, x, 0.2 * x)


def _pass1_kernel(M, R, T,
                  x_ref, x0_ref, wg1_ref, asp_ref, adp_ref, eb_ref, ef_ref,
                  bg1_ref, wg2_ref, os2_ref, as2_ref, ad2_ref,
                  xw2_ref, xw20_ref, scal_ref,
                  m1_ref, s1_ref, acc1_ref, mals2_ref):
    i = pl.program_id(0)
    bf16 = jnp.bfloat16

    @pl.when(i == 0)
    def _init():
        m1_ref[...] = jnp.full((1, 128), _NEG, jnp.float32)
        s1_ref[...] = jnp.zeros((1, 128), jnp.float32)
        acc1_ref[...] = jnp.zeros((1, 512), jnp.float32)
        mals2_ref[...] = jnp.full((1, 128), _NEG, jnp.float32)

    base = i * R
    rid = jax.lax.broadcasted_iota(jnp.int32, (R, 128), 0)
    valid = (base + rid) < M

    Xt = jnp.where(valid, x_ref[...], bf16(0))
    XW = jnp.dot(Xt, wg1_ref[...], preferred_element_type=jnp.float32)
    XWb = XW.astype(bf16)
    ALS = jnp.dot(XWb, asp_ref[...], preferred_element_type=jnp.float32)
    ALD = jnp.dot(XWb, adp_ref[...], preferred_element_type=jnp.float32)

    xw0 = jnp.dot(x0_ref[...], wg1_ref[...], preferred_element_type=jnp.float32)
    xw0b = xw0.astype(bf16)
    als0 = jnp.dot(xw0b, asp_ref[...], preferred_element_type=jnp.float32)
    ald0 = jnp.dot(xw0b, adp_ref[...], preferred_element_type=jnp.float32)

    e0j = _lrelu(als0 + ALD)
    ejj = _lrelu(ALS + ALD)
    aj = jax.nn.sigmoid(ejj - e0j)
    Aje = jnp.dot(aj.astype(bf16), eb_ref[...], preferred_element_type=jnp.float32)
    X1 = xw0 + Aje * (XW - xw0)
    X1 = jnp.maximum(X1 + bg1_ref[...], 0.0)

    ej0 = jnp.where(valid, _lrelu(ALS + ald0), _NEG)
    tm = jnp.max(ej0, axis=0, keepdims=True)
    mold = m1_ref[...]
    mnew = jnp.maximum(mold, tm)
    scale = jnp.exp(mold - mnew)
    w = jnp.exp(ej0 - mnew)
    s1_ref[...] = s1_ref[...] * scale + jnp.sum(w, axis=0, keepdims=True)
    wE = jnp.dot(w.astype(bf16), eb_ref[...], preferred_element_type=jnp.float32)
    scE = jnp.dot(scale, ef_ref[...], preferred_element_type=jnp.float32)
    acc1_ref[...] = acc1_ref[...] * scE + jnp.sum(wE * XW, axis=0, keepdims=True)
    m1_ref[...] = mnew

    XW2 = jnp.dot(X1.astype(bf16), wg2_ref[...], preferred_element_type=jnp.float32)
    XW2 = jnp.where(valid, XW2, 0.0)
    XW2b = XW2.astype(bf16)
    xw2_ref[...] = XW2b
    ALS2 = jnp.dot(XW2b, os2_ref[...], preferred_element_type=jnp.float32)
    ALS2 = jnp.where(valid, ALS2, _NEG)
    mals2_ref[...] = jnp.maximum(mals2_ref[...], jnp.max(ALS2, axis=0, keepdims=True))

    @pl.when(i == T - 1)
    def _epilogue():
        e00 = _lrelu(als0 + ald0)
        mo = m1_ref[...]
        mf = jnp.maximum(mo, e00)
        sc_o = jnp.exp(mo - mf)
        sc_s = jnp.exp(e00 - mf)
        s = s1_ref[...] * sc_o + sc_s
        accf = (acc1_ref[...] * jnp.dot(sc_o, ef_ref[...], preferred_element_type=jnp.float32)
                + jnp.dot(sc_s, ef_ref[...], preferred_element_type=jnp.float32) * xw0)
        sE = jnp.dot(s, ef_ref[...], preferred_element_type=jnp.float32)
        x1_0 = jnp.maximum(accf / sE + bg1_ref[...], 0.0)
        xw2_0 = jnp.dot(x1_0.astype(bf16), wg2_ref[...],
                        preferred_element_type=jnp.float32)
        xw20_ref[...] = xw2_0
        als2_0 = jnp.sum(xw2_0 * as2_ref[...])
        ald2_0 = jnp.sum(xw2_0 * ad2_ref[...])
        gmax = jnp.maximum(jnp.max(mals2_ref[...]), als2_0)
        lane = jax.lax.broadcasted_iota(jnp.int32, (1, 128), 1)
        scal_ref[...] = (jnp.where(lane == 0, als2_0, 0.0)
                         + jnp.where(lane == 1, ald2_0, 0.0)
                         + jnp.where(lane == 2, gmax, 0.0))


def _pass2_kernel(M, R, T,
                  xw2_ref, xw20_ref, scal_ref, os2_ref, bg2_ref,
                  qe_ref, wq1_ref, bq1_ref, wq2_ref, bq2_ref,
                  pe_ref, pr_ref, wep_ref, bep_ref,
                  wihe_ref, wihr_ref, bih_ref, whh_ref, bhh_ref,
                  wqp_ref, bqp_ref, wpp_ref, bpp_ref, wef_ref, bef_ref,
                  wp1a_ref, wp1b_ref, wp1c_ref, bp1_ref,
                  wp2_ref, bp2_ref, wp3_ref, bp3_ref, vr_ref,
                  probs_ref, vlog_ref,
                  s2_ref, acc2_ref):
    i = pl.program_id(0)

    @pl.when(i == 0)
    def _init():
        s2_ref[...] = jnp.zeros((1, 128), jnp.float32)
        acc2_ref[...] = jnp.zeros((1, 128), jnp.float32)

    lane = jax.lax.broadcasted_iota(jnp.int32, (1, 128), 1)
    scal = scal_ref[...]
    als2_0 = jnp.sum(jnp.where(lane == 0, scal, 0.0))
    ald2_0 = jnp.sum(jnp.where(lane == 1, scal, 0.0))
    gmax = jnp.sum(jnp.where(lane == 2, scal, 0.0))
    m2 = _lrelu(gmax + ald2_0)

    XW2b = xw2_ref[...]
    XW2 = XW2b.astype(jnp.float32)
    rid = jax.lax.broadcasted_iota(jnp.int32, (R, 128), 0)
    valid = (i * R + rid) < M
    XW2 = jnp.where(valid, XW2, 0.0)
    ALS2 = jnp.dot(XW2b, os2_ref[...], preferred_element_type=jnp.float32)
    e2 = jnp.where(valid, _lrelu(ALS2 + ald2_0), _NEG)
    w = jnp.exp(e2 - m2)
    acc2_ref[...] = acc2_ref[...] + jnp.sum(w * XW2, axis=0, keepdims=True)
    s2_ref[...] = s2_ref[...] + jnp.sum(w, axis=0, keepdims=True)

    @pl.when(i == T - 1)
    def _epilogue():
        e00 = _lrelu(als2_0 + ald2_0)
        w00 = jnp.exp(e00 - m2)
        acc = acc2_ref[...] + w00 * xw20_ref[...]
        s = s2_ref[...] + w00
        ent = jnp.maximum(acc / s + bg2_ref[...], 0.0)

        q = jnp.maximum(jnp.dot(qe_ref[...], wq1_ref[...],
                                preferred_element_type=jnp.float32) + bq1_ref[...], 0.0)
        q = jnp.dot(q, wq2_ref[...], preferred_element_type=jnp.float32) + bq2_ref[...]

        ents = jnp.dot(pe_ref[...], wep_ref[...],
                       preferred_element_type=jnp.float32) + bep_ref[...]
        prel = pr_ref[...]
        h = jnp.zeros((1, 128), jnp.float32)
        c = jnp.zeros((1, 128), jnp.float32)
        for t in range(5):
            g = (jnp.dot(ents[t:t + 1, :], wihe_ref[...], preferred_element_type=jnp.float32)
                 + jnp.dot(prel[t:t + 1, :], wihr_ref[...], preferred_element_type=jnp.float32)
                 + bih_ref[...]
                 + jnp.dot(h, whh_ref[...], preferred_element_type=jnp.float32)
                 + bhh_ref[...])
            ig = jax.nn.sigmoid(g[:, 0:128])
            fg = jax.nn.sigmoid(g[:, 128:256])
            gg = jnp.tanh(g[:, 256:384])
            og = jax.nn.sigmoid(g[:, 384:512])
            c = fg * c + ig * gg
            h = og * jnp.tanh(c)

        qp = jnp.dot(q, wqp_ref[...], preferred_element_type=jnp.float32) + bqp_ref[...]
        pp = jnp.dot(h, wpp_ref[...], preferred_element_type=jnp.float32) + bpp_ref[...]
        ep = jnp.dot(ent, wef_ref[...], preferred_element_type=jnp.float32) + bef_ref[...]
        hh = jnp.maximum(jnp.dot(qp, wp1a_ref[...], preferred_element_type=jnp.float32)
                         + jnp.dot(pp, wp1b_ref[...], preferred_element_type=jnp.float32)
                         + jnp.dot(ep, wp1c_ref[...], preferred_element_type=jnp.float32)
                         + bp1_ref[...], 0.0)
        h2 = jnp.maximum(jnp.dot(hh, wp2_ref[...], preferred_element_type=jnp.float32)
                         + bp2_ref[...], 0.0)
        logits = jnp.dot(h2, wp3_ref[...], preferred_element_type=jnp.float32) + bp3_ref[...]

        vr = vr_ref[...]
        oh = (jax.lax.broadcasted_iota(jnp.int32, (1024, 64), 0) == vr).astype(jnp.float32)
        vl = jnp.dot(logits, oh, preferred_element_type=jnp.float32)
        mx = jnp.max(vl)
        ex = jnp.exp(vl - mx)
        probs_ref[...] = ex / jnp.sum(ex)
        vlog_ref[...] = vl


def kernel(question_emb, current_entity_emb, path_entities, path_relations,
           neighbor_entities, valid_relations,
           Wq1, bq1, Wq2, bq2, Wep, bep, W_ih, W_hh, b_ih, b_hh,
           Wg1, as1, ad1, bg1, Wg2, as2, ad2, bg2,
           Wqp, bqp, Wpp, bpp, Wef, bef, Wp1, bp1, Wp2, bp2, Wp3, bp3):
    f32 = jnp.float32
    bf16 = jnp.bfloat16
    M = neighbor_entities.shape[0]
    R1 = 2048
    T1 = (M + R1 - 1) // R1
    R2 = 8192
    T2 = (T1 * R1 + R2 - 1) // R2
    NREL = Wp3.shape[1]
    NRELP = ((NREL + 127) // 128) * 128
    NV = valid_relations.shape[0]

    idx = jnp.arange(512)
    hcol = idx // 128
    asp = jnp.zeros((512, 128), f32).at[idx, hcol].set(as1.reshape(-1)).astype(bf16)
    adp = jnp.zeros((512, 128), f32).at[idx, hcol].set(ad1.reshape(-1)).astype(bf16)
    Ef = (hcol[None, :] == jnp.arange(128)[:, None]).astype(f32)
    Eb = Ef.astype(bf16)
    Os2 = jnp.broadcast_to(as2.reshape(128, 1), (128, 128)).astype(bf16)

    x0r = current_entity_emb.reshape(1, 128).astype(bf16)
    row = lambda v: v.reshape(1, -1)

    full = lambda shp: pl.BlockSpec(shp, lambda i: tuple(0 for _ in shp))
    p1 = pl.pallas_call(
        functools.partial(_pass1_kernel, M, R1, T1),
        grid=(T1,),
        in_specs=[
            pl.BlockSpec((R1, 128), lambda i: (i, 0)),
            full((1, 128)),
            full((128, 512)),
            full((512, 128)),
            full((512, 128)),
            full((128, 512)),
            full((128, 512)),
            full((1, 512)),
            full((512, 128)),
            full((128, 128)),
            full((1, 128)),
            full((1, 128)),
        ],
        out_specs=[
            pl.BlockSpec((R1, 128), lambda i: (i, 0)),
            full((1, 128)),
            full((1, 128)),
        ],
        out_shape=[
            jax.ShapeDtypeStruct((T1 * R1, 128), bf16),
            jax.ShapeDtypeStruct((1, 128), f32),
            jax.ShapeDtypeStruct((1, 128), f32),
        ],
        scratch_shapes=[
            pltpu.VMEM((1, 128), f32),
            pltpu.VMEM((1, 128), f32),
            pltpu.VMEM((1, 512), f32),
            pltpu.VMEM((1, 128), f32),
        ],
    )
    xw2_buf, xw2_0, scal = p1(
        neighbor_entities.astype(bf16), x0r, Wg1.astype(bf16), asp, adp,
        Eb, Ef, row(bg1), Wg2.astype(bf16), Os2, as2, ad2)

    wp3p = jnp.zeros((Wp3.shape[0], NRELP), f32).at[:, :NREL].set(Wp3)
    bp3p = jnp.zeros((1, NRELP), f32).at[0, :NREL].set(bp3)

    p2 = pl.pallas_call(
        functools.partial(_pass2_kernel, M, R2, T2),
        grid=(T2,),
        in_specs=[
            pl.BlockSpec((R2, 128), lambda i: (i, 0)),
            full((1, 128)),
            full((1, 128)),
            full((128, 128)),
            full((1, 128)),
            full((1, 128)),
            full((128, 128)), full((1, 128)),
            full((128, 128)), full((1, 128)),
            full((5, 128)),
            full((5, 128)),
            full((128, 128)), full((1, 128)),
            full((128, 512)),
            full((128, 512)),
            full((1, 512)),
            full((128, 512)),
            full((1, 512)),
            full((128, 128)), full((1, 128)),
            full((128, 128)), full((1, 128)),
            full((128, 128)), full((1, 128)),
            full((128, 128)), full((128, 128)), full((128, 128)),
            full((1, 128)),
            full((128, 64)), full((1, 64)),
            full((64, NRELP)), full((1, NRELP)),
            full((1, NV)),
        ],
        out_specs=[full((1, NV)), full((1, NV))],
        out_shape=[
            jax.ShapeDtypeStruct((1, NV), f32),
            jax.ShapeDtypeStruct((1, NV), f32),
        ],
        scratch_shapes=[
            pltpu.VMEM((1, 128), f32),
            pltpu.VMEM((1, 128), f32),
        ],
    )
    probs, vlog = p2(
        xw2_buf, xw2_0, scal, Os2, row(bg2),
        row(question_emb), Wq1, row(bq1), Wq2, row(bq2),
        path_entities[:-1], path_relations, Wep, row(bep),
        W_ih[:, :128].T, W_ih[:, 128:].T, row(b_ih), W_hh.T, row(b_hh),
        Wqp, row(bqp), Wpp, row(bpp), Wef, row(bef),
        Wp1[0:128], Wp1[128:256], Wp1[256:384], row(bp1),
        Wp2, row(bp2), wp3p, bp3p, valid_relations.reshape(1, -1),
    )
    return probs.reshape(-1), vlog.reshape(-1)


# speedup vs baseline: 1.1654x; 1.0215x over previous
import functools

import jax
import jax.numpy as jnp
from jax.experimental import pallas as pl
from jax.experimental.pallas import tpu as pltpu

_NEG = -1e30


def _lrelu(x):
    return jnp.where(x > 0, x, 0.2 * x)


def _pass1_kernel(M, R, T,
                  x_ref, x0_ref, wg1_ref, asp_ref, adp_ref, eb_ref, ef_ref,
                  bg1_ref, wg2_ref, os2_ref, as2_ref, ad2_ref,
                  xw2_ref, xw20_ref, scal_ref,
                  m1_ref, s1_ref, acc1_ref, mals2_ref):
    i = pl.program_id(0)
    bf16 = jnp.bfloat16

    @pl.when(i == 0)
    def _init():
        m1_ref[...] = jnp.full((1, 128), _NEG, jnp.float32)
        s1_ref[...] = jnp.zeros((1, 128), jnp.float32)
        acc1_ref[...] = jnp.zeros((1, 512), jnp.float32)
        mals2_ref[...] = jnp.full((1, 128), _NEG, jnp.float32)

    base = i * R
    rid = jax.lax.broadcasted_iota(jnp.int32, (R, 128), 0)
    valid = (base + rid) < M

    Xt = jnp.where(valid, x_ref[...], bf16(0))
    XW = jnp.dot(Xt, wg1_ref[...], preferred_element_type=jnp.float32)
    XWb = XW.astype(bf16)
    ALS = jnp.dot(XWb, asp_ref[...], preferred_element_type=jnp.float32)
    ALD = jnp.dot(XWb, adp_ref[...], preferred_element_type=jnp.float32)

    xw0 = jnp.dot(x0_ref[...], wg1_ref[...], preferred_element_type=jnp.float32)
    xw0b = xw0.astype(bf16)
    als0 = jnp.dot(xw0b, asp_ref[...], preferred_element_type=jnp.float32)
    ald0 = jnp.dot(xw0b, adp_ref[...], preferred_element_type=jnp.float32)

    e0j = _lrelu(als0 + ALD)
    ejj = _lrelu(ALS + ALD)
    aj = jax.nn.sigmoid(ejj - e0j)
    Aje = jnp.dot(aj.astype(bf16), eb_ref[...], preferred_element_type=jnp.float32)
    X1 = xw0 + Aje * (XW - xw0)
    X1 = jnp.maximum(X1 + bg1_ref[...], 0.0).astype(bf16)

    ej0 = jnp.where(valid, _lrelu(ALS + ald0), _NEG)
    tm = jnp.max(ej0, axis=0, keepdims=True)
    mold = m1_ref[...]
    mnew = jnp.maximum(mold, tm)
    scale = jnp.exp(mold - mnew)
    w = jnp.exp(ej0 - mnew)
    s1_ref[...] = s1_ref[...] * scale + jnp.sum(w, axis=0, keepdims=True)
    C = jax.lax.dot_general(w.astype(bf16), XWb, (((0,), (0,)), ((), ())),
                            preferred_element_type=jnp.float32)
    Crow = jnp.concatenate(
        [C[h:h + 1, 128 * h:128 * (h + 1)] for h in range(4)], axis=1)
    scE = jnp.dot(scale, ef_ref[...], preferred_element_type=jnp.float32)
    acc1_ref[...] = acc1_ref[...] * scE + Crow
    m1_ref[...] = mnew

    XW2 = jnp.dot(X1, wg2_ref[...], preferred_element_type=jnp.float32)
    XW2 = jnp.where(valid, XW2, 0.0)
    XW2b = XW2.astype(bf16)
    xw2_ref[...] = XW2b
    ALS2 = jnp.dot(XW2b, os2_ref[...], preferred_element_type=jnp.float32)
    ALS2 = jnp.where(valid, ALS2, _NEG)
    mals2_ref[...] = jnp.maximum(mals2_ref[...], jnp.max(ALS2, axis=0, keepdims=True))

    @pl.when(i == T - 1)
    def _epilogue():
        e00 = _lrelu(als0 + ald0)
        mo = m1_ref[...]
        mf = jnp.maximum(mo, e00)
        sc_o = jnp.exp(mo - mf)
        sc_s = jnp.exp(e00 - mf)
        s = s1_ref[...] * sc_o + sc_s
        accf = (acc1_ref[...] * jnp.dot(sc_o, ef_ref[...], preferred_element_type=jnp.float32)
                + jnp.dot(sc_s, ef_ref[...], preferred_element_type=jnp.float32) * xw0)
        sE = jnp.dot(s, ef_ref[...], preferred_element_type=jnp.float32)
        x1_0 = jnp.maximum(accf / sE + bg1_ref[...], 0.0)
        xw2_0 = jnp.dot(x1_0.astype(bf16), wg2_ref[...],
                        preferred_element_type=jnp.float32)
        xw20_ref[...] = xw2_0
        als2_0 = jnp.sum(xw2_0 * as2_ref[...])
        ald2_0 = jnp.sum(xw2_0 * ad2_ref[...])
        gmax = jnp.maximum(jnp.max(mals2_ref[...]), als2_0)
        lane = jax.lax.broadcasted_iota(jnp.int32, (1, 128), 1)
        scal_ref[...] = (jnp.where(lane == 0, als2_0, 0.0)
                         + jnp.where(lane == 1, ald2_0, 0.0)
                         + jnp.where(lane == 2, gmax, 0.0))


def _pass2_kernel(M, R, T,
                  xw2_ref, xw20_ref, scal_ref, os2_ref, bg2_ref,
                  qe_ref, wq1_ref, bq1_ref, wq2_ref, bq2_ref,
                  pe_ref, pr_ref, wep_ref, bep_ref,
                  wihe_ref, wihr_ref, bih_ref, whh_ref, bhh_ref,
                  wqp_ref, bqp_ref, wpp_ref, bpp_ref, wef_ref, bef_ref,
                  wp1a_ref, wp1b_ref, wp1c_ref, bp1_ref,
                  wp2_ref, bp2_ref, wp3_ref, bp3_ref, vr_ref,
                  probs_ref, vlog_ref,
                  s2_ref, acc2_ref):
    i = pl.program_id(0)

    @pl.when(i == 0)
    def _init():
        s2_ref[...] = jnp.zeros((1, 128), jnp.float32)
        acc2_ref[...] = jnp.zeros((1, 128), jnp.float32)

    lane = jax.lax.broadcasted_iota(jnp.int32, (1, 128), 1)
    scal = scal_ref[...]
    als2_0 = jnp.sum(jnp.where(lane == 0, scal, 0.0))
    ald2_0 = jnp.sum(jnp.where(lane == 1, scal, 0.0))
    gmax = jnp.sum(jnp.where(lane == 2, scal, 0.0))
    m2 = _lrelu(gmax + ald2_0)

    XW2b = xw2_ref[...]
    XW2 = XW2b.astype(jnp.float32)
    rid = jax.lax.broadcasted_iota(jnp.int32, (R, 128), 0)
    valid = (i * R + rid) < M
    XW2 = jnp.where(valid, XW2, 0.0)
    ALS2 = jnp.dot(XW2b, os2_ref[...], preferred_element_type=jnp.float32)
    e2 = jnp.where(valid, _lrelu(ALS2 + ald2_0), _NEG)
    w = jnp.exp(e2 - m2)
    acc2_ref[...] = acc2_ref[...] + jnp.sum(w * XW2, axis=0, keepdims=True)
    s2_ref[...] = s2_ref[...] + jnp.sum(w, axis=0, keepdims=True)

    @pl.when(i == T - 1)
    def _epilogue():
        e00 = _lrelu(als2_0 + ald2_0)
        w00 = jnp.exp(e00 - m2)
        acc = acc2_ref[...] + w00 * xw20_ref[...]
        s = s2_ref[...] + w00
        ent = jnp.maximum(acc / s + bg2_ref[...], 0.0)

        q = jnp.maximum(jnp.dot(qe_ref[...], wq1_ref[...],
                                preferred_element_type=jnp.float32) + bq1_ref[...], 0.0)
        q = jnp.dot(q, wq2_ref[...], preferred_element_type=jnp.float32) + bq2_ref[...]

        ents = jnp.dot(pe_ref[...], wep_ref[...],
                       preferred_element_type=jnp.float32) + bep_ref[...]
        prel = pr_ref[...]
        h = jnp.zeros((1, 128), jnp.float32)
        c = jnp.zeros((1, 128), jnp.float32)
        for t in range(5):
            g = (jnp.dot(ents[t:t + 1, :], wihe_ref[...], preferred_element_type=jnp.float32)
                 + jnp.dot(prel[t:t + 1, :], wihr_ref[...], preferred_element_type=jnp.float32)
                 + bih_ref[...]
                 + jnp.dot(h, whh_ref[...], preferred_element_type=jnp.float32)
                 + bhh_ref[...])
            ig = jax.nn.sigmoid(g[:, 0:128])
            fg = jax.nn.sigmoid(g[:, 128:256])
            gg = jnp.tanh(g[:, 256:384])
            og = jax.nn.sigmoid(g[:, 384:512])
            c = fg * c + ig * gg
            h = og * jnp.tanh(c)

        qp = jnp.dot(q, wqp_ref[...], preferred_element_type=jnp.float32) + bqp_ref[...]
        pp = jnp.dot(h, wpp_ref[...], preferred_element_type=jnp.float32) + bpp_ref[...]
        ep = jnp.dot(ent, wef_ref[...], preferred_element_type=jnp.float32) + bef_ref[...]
        hh = jnp.maximum(jnp.dot(qp, wp1a_ref[...], preferred_element_type=jnp.float32)
                         + jnp.dot(pp, wp1b_ref[...], preferred_element_type=jnp.float32)
                         + jnp.dot(ep, wp1c_ref[...], preferred_element_type=jnp.float32)
                         + bp1_ref[...], 0.0)
        h2 = jnp.maximum(jnp.dot(hh, wp2_ref[...], preferred_element_type=jnp.float32)
                         + bp2_ref[...], 0.0)
        logits = jnp.dot(h2, wp3_ref[...], preferred_element_type=jnp.float32) + bp3_ref[...]

        vr = vr_ref[...]
        oh = (jax.lax.broadcasted_iota(jnp.int32, (1024, 64), 0) == vr).astype(jnp.float32)
        vl = jnp.dot(logits, oh, preferred_element_type=jnp.float32)
        mx = jnp.max(vl)
        ex = jnp.exp(vl - mx)
        probs_ref[...] = ex / jnp.sum(ex)
        vlog_ref[...] = vl


def kernel(question_emb, current_entity_emb, path_entities, path_relations,
           neighbor_entities, valid_relations,
           Wq1, bq1, Wq2, bq2, Wep, bep, W_ih, W_hh, b_ih, b_hh,
           Wg1, as1, ad1, bg1, Wg2, as2, ad2, bg2,
           Wqp, bqp, Wpp, bpp, Wef, bef, Wp1, bp1, Wp2, bp2, Wp3, bp3):
    f32 = jnp.float32
    bf16 = jnp.bfloat16
    M = neighbor_entities.shape[0]
    R1 = 4096
    T1 = (M + R1 - 1) // R1
    R2 = 8192
    T2 = (T1 * R1 + R2 - 1) // R2
    NREL = Wp3.shape[1]
    NRELP = ((NREL + 127) // 128) * 128
    NV = valid_relations.shape[0]

    idx = jnp.arange(512)
    hcol = idx // 128
    asp = jnp.zeros((512, 128), f32).at[idx, hcol].set(as1.reshape(-1)).astype(bf16)
    adp = jnp.zeros((512, 128), f32).at[idx, hcol].set(ad1.reshape(-1)).astype(bf16)
    Ef = (hcol[None, :] == jnp.arange(128)[:, None]).astype(f32)
    Eb = Ef.astype(bf16)
    Os2 = jnp.broadcast_to(as2.reshape(128, 1), (128, 128)).astype(bf16)

    x0r = current_entity_emb.reshape(1, 128).astype(bf16)
    row = lambda v: v.reshape(1, -1)

    full = lambda shp: pl.BlockSpec(shp, lambda i: tuple(0 for _ in shp))
    p1 = pl.pallas_call(
        functools.partial(_pass1_kernel, M, R1, T1),
        grid=(T1,),
        in_specs=[
            pl.BlockSpec((R1, 128), lambda i: (i, 0)),
            full((1, 128)),
            full((128, 512)),
            full((512, 128)),
            full((512, 128)),
            full((128, 512)),
            full((128, 512)),
            full((1, 512)),
            full((512, 128)),
            full((128, 128)),
            full((1, 128)),
            full((1, 128)),
        ],
        out_specs=[
            pl.BlockSpec((R1, 128), lambda i: (i, 0)),
            full((1, 128)),
            full((1, 128)),
        ],
        out_shape=[
            jax.ShapeDtypeStruct((T1 * R1, 128), bf16),
            jax.ShapeDtypeStruct((1, 128), f32),
            jax.ShapeDtypeStruct((1, 128), f32),
        ],
        scratch_shapes=[
            pltpu.VMEM((1, 128), f32),
            pltpu.VMEM((1, 128), f32),
            pltpu.VMEM((1, 512), f32),
            pltpu.VMEM((1, 128), f32),
        ],
    )
    xw2_buf, xw2_0, scal = p1(
        neighbor_entities.astype(bf16), x0r, Wg1.astype(bf16), asp, adp,
        Eb, Ef, row(bg1).astype(bf16), Wg2.astype(bf16), Os2, as2, ad2)

    wp3p = jnp.zeros((Wp3.shape[0], NRELP), f32).at[:, :NREL].set(Wp3)
    bp3p = jnp.zeros((1, NRELP), f32).at[0, :NREL].set(bp3)

    p2 = pl.pallas_call(
        functools.partial(_pass2_kernel, M, R2, T2),
        grid=(T2,),
        in_specs=[
            pl.BlockSpec((R2, 128), lambda i: (i, 0)),
            full((1, 128)),
            full((1, 128)),
            full((128, 128)),
            full((1, 128)),
            full((1, 128)),
            full((128, 128)), full((1, 128)),
            full((128, 128)), full((1, 128)),
            full((5, 128)),
            full((5, 128)),
            full((128, 128)), full((1, 128)),
            full((128, 512)),
            full((128, 512)),
            full((1, 512)),
            full((128, 512)),
            full((1, 512)),
            full((128, 128)), full((1, 128)),
            full((128, 128)), full((1, 128)),
            full((128, 128)), full((1, 128)),
            full((128, 128)), full((128, 128)), full((128, 128)),
            full((1, 128)),
            full((128, 64)), full((1, 64)),
            full((64, NRELP)), full((1, NRELP)),
            full((1, NV)),
        ],
        out_specs=[full((1, NV)), full((1, NV))],
        out_shape=[
            jax.ShapeDtypeStruct((1, NV), f32),
            jax.ShapeDtypeStruct((1, NV), f32),
        ],
        scratch_shapes=[
            pltpu.VMEM((1, 128), f32),
            pltpu.VMEM((1, 128), f32),
        ],
    )
    probs, vlog = p2(
        xw2_buf, xw2_0, scal, Os2, row(bg2),
        row(question_emb), Wq1, row(bq1), Wq2, row(bq2),
        path_entities[:-1], path_relations, Wep, row(bep),
        W_ih[:, :128].T, W_ih[:, 128:].T, row(b_ih), W_hh.T, row(b_hh),
        Wqp, row(bqp), Wpp, row(bpp), Wef, row(bef),
        Wp1[0:128], Wp1[128:256], Wp1[256:384], row(bp1),
        Wp2, row(bp2), wp3p, bp3p, valid_relations.reshape(1, -1),
    )
    return probs.reshape(-1), vlog.reshape(-1)


# speedup vs baseline: 1.2173x; 1.0446x over previous
import functools

import jax
import jax.numpy as jnp
from jax.experimental import pallas as pl
from jax.experimental.pallas import tpu as pltpu

_NEG = -1e30


def _lrelu(x):
    return jnp.where(x > 0, x, 0.2 * x)


def _pass1_kernel(M, R, T,
                  x_ref, x0_ref, wg1_ref, asp_ref, adp_ref, eb_ref, ef_ref,
                  bg1_ref, wg2_ref, os2_ref, as2_ref, ad2_ref,
                  xw2_ref, xw20_ref, scal_ref,
                  m1_ref, s1_ref, acc1_ref, mals2_ref):
    i = pl.program_id(0)
    bf16 = jnp.bfloat16

    @pl.when(i == 0)
    def _init():
        m1_ref[...] = jnp.full((1, 128), _NEG, jnp.float32)
        s1_ref[...] = jnp.zeros((1, 128), jnp.float32)
        acc1_ref[...] = jnp.zeros((1, 512), jnp.float32)
        mals2_ref[...] = jnp.full((1, 128), _NEG, jnp.float32)

    base = i * R
    rid = jax.lax.broadcasted_iota(jnp.int32, (R, 128), 0)
    valid = (base + rid) < M

    Xt = jnp.where(valid, x_ref[...], 0.0).astype(bf16)
    XW = jnp.dot(Xt, wg1_ref[...], preferred_element_type=jnp.float32)
    XWb = XW.astype(bf16)
    ALS = jnp.dot(XWb, asp_ref[...], preferred_element_type=jnp.float32)
    ALD = jnp.dot(XWb, adp_ref[...], preferred_element_type=jnp.float32)

    xw0 = jnp.dot(x0_ref[...], wg1_ref[...], preferred_element_type=jnp.float32)
    xw0b = xw0.astype(bf16)
    als0 = jnp.dot(xw0b, asp_ref[...], preferred_element_type=jnp.float32)
    ald0 = jnp.dot(xw0b, adp_ref[...], preferred_element_type=jnp.float32)

    e0j = _lrelu(als0 + ALD)
    ejj = _lrelu(ALS + ALD)
    aj = jax.nn.sigmoid(ejj - e0j)
    Aje = jnp.dot(aj.astype(bf16), eb_ref[...], preferred_element_type=jnp.float32)
    X1 = xw0 + Aje * (XW - xw0)
    X1 = jnp.maximum(X1 + bg1_ref[...], 0.0).astype(bf16)

    ej0 = jnp.where(valid, _lrelu(ALS + ald0), _NEG)
    tm = jnp.max(ej0, axis=0, keepdims=True)
    mold = m1_ref[...]
    mnew = jnp.maximum(mold, tm)
    scale = jnp.exp(mold - mnew)
    w = jnp.exp(ej0 - mnew)
    s1_ref[...] = s1_ref[...] * scale + jnp.sum(w, axis=0, keepdims=True)
    C = jax.lax.dot_general(w.astype(bf16), XWb, (((0,), (0,)), ((), ())),
                            preferred_element_type=jnp.float32)
    Crow = jnp.concatenate(
        [C[h:h + 1, 128 * h:128 * (h + 1)] for h in range(4)], axis=1)
    scE = jnp.dot(scale, ef_ref[...], preferred_element_type=jnp.float32)
    acc1_ref[...] = acc1_ref[...] * scE + Crow
    m1_ref[...] = mnew

    XW2 = jnp.dot(X1, wg2_ref[...], preferred_element_type=jnp.float32)
    XW2 = jnp.where(valid, XW2, 0.0)
    XW2b = XW2.astype(bf16)
    xw2_ref[...] = XW2b
    ALS2 = jnp.dot(XW2b, os2_ref[...], preferred_element_type=jnp.float32)
    ALS2 = jnp.where(valid, ALS2, _NEG)
    mals2_ref[...] = jnp.maximum(mals2_ref[...], jnp.max(ALS2, axis=0, keepdims=True))

    @pl.when(i == T - 1)
    def _epilogue():
        e00 = _lrelu(als0 + ald0)
        mo = m1_ref[...]
        mf = jnp.maximum(mo, e00)
        sc_o = jnp.exp(mo - mf)
        sc_s = jnp.exp(e00 - mf)
        s = s1_ref[...] * sc_o + sc_s
        accf = (acc1_ref[...] * jnp.dot(sc_o, ef_ref[...], preferred_element_type=jnp.float32)
                + jnp.dot(sc_s, ef_ref[...], preferred_element_type=jnp.float32) * xw0)
        sE = jnp.dot(s, ef_ref[...], preferred_element_type=jnp.float32)
        x1_0 = jnp.maximum(accf / sE + bg1_ref[...], 0.0)
        xw2_0 = jnp.dot(x1_0.astype(bf16), wg2_ref[...],
                        preferred_element_type=jnp.float32)
        xw20_ref[...] = xw2_0
        als2_0 = jnp.sum(xw2_0 * as2_ref[...])
        ald2_0 = jnp.sum(xw2_0 * ad2_ref[...])
        gmax = jnp.maximum(jnp.max(mals2_ref[...]), als2_0)
        lane = jax.lax.broadcasted_iota(jnp.int32, (1, 128), 1)
        scal_ref[...] = (jnp.where(lane == 0, als2_0, 0.0)
                         + jnp.where(lane == 1, ald2_0, 0.0)
                         + jnp.where(lane == 2, gmax, 0.0))


def _pass2_kernel(M, R, T,
                  xw2_ref, xw20_ref, scal_ref, os2_ref, bg2_ref,
                  qe_ref, wq1_ref, bq1_ref, wq2_ref, bq2_ref,
                  pe_ref, pr_ref, wep_ref, bep_ref,
                  wihe_ref, wihr_ref, bih_ref, whh_ref, bhh_ref,
                  wqp_ref, bqp_ref, wpp_ref, bpp_ref, wef_ref, bef_ref,
                  wp1a_ref, wp1b_ref, wp1c_ref, bp1_ref,
                  wp2_ref, bp2_ref, wp3_ref, bp3_ref, vr_ref,
                  probs_ref, vlog_ref,
                  s2_ref, acc2_ref):
    i = pl.program_id(0)

    @pl.when(i == 0)
    def _init():
        s2_ref[...] = jnp.zeros((1, 128), jnp.float32)
        acc2_ref[...] = jnp.zeros((1, 128), jnp.float32)

    lane = jax.lax.broadcasted_iota(jnp.int32, (1, 128), 1)
    scal = scal_ref[...]
    als2_0 = jnp.sum(jnp.where(lane == 0, scal, 0.0))
    ald2_0 = jnp.sum(jnp.where(lane == 1, scal, 0.0))
    gmax = jnp.sum(jnp.where(lane == 2, scal, 0.0))
    m2 = _lrelu(gmax + ald2_0)

    XW2b = xw2_ref[...]
    XW2 = XW2b.astype(jnp.float32)
    rid = jax.lax.broadcasted_iota(jnp.int32, (R, 128), 0)
    valid = (i * R + rid) < M
    XW2 = jnp.where(valid, XW2, 0.0)
    ALS2 = jnp.dot(XW2b, os2_ref[...], preferred_element_type=jnp.float32)
    e2 = jnp.where(valid, _lrelu(ALS2 + ald2_0), _NEG)
    w = jnp.exp(e2 - m2)
    acc2_ref[...] = acc2_ref[...] + jnp.sum(w * XW2, axis=0, keepdims=True)
    s2_ref[...] = s2_ref[...] + jnp.sum(w, axis=0, keepdims=True)

    @pl.when(i == T - 1)
    def _epilogue():
        e00 = _lrelu(als2_0 + ald2_0)
        w00 = jnp.exp(e00 - m2)
        acc = acc2_ref[...] + w00 * xw20_ref[...]
        s = s2_ref[...] + w00
        ent = jnp.maximum(acc / s + bg2_ref[...], 0.0)

        q = jnp.maximum(jnp.dot(qe_ref[...], wq1_ref[...],
                                preferred_element_type=jnp.float32) + bq1_ref[...], 0.0)
        q = jnp.dot(q, wq2_ref[...], preferred_element_type=jnp.float32) + bq2_ref[...]

        ents = jnp.dot(pe_ref[...], wep_ref[...],
                       preferred_element_type=jnp.float32) + bep_ref[...]
        prel = pr_ref[...]
        h = jnp.zeros((1, 128), jnp.float32)
        c = jnp.zeros((1, 128), jnp.float32)
        for t in range(5):
            g = (jnp.dot(ents[t:t + 1, :], wihe_ref[...], preferred_element_type=jnp.float32)
                 + jnp.dot(prel[t:t + 1, :], wihr_ref[...], preferred_element_type=jnp.float32)
                 + bih_ref[...]
                 + jnp.dot(h, whh_ref[...], preferred_element_type=jnp.float32)
                 + bhh_ref[...])
            ig = jax.nn.sigmoid(g[:, 0:128])
            fg = jax.nn.sigmoid(g[:, 128:256])
            gg = jnp.tanh(g[:, 256:384])
            og = jax.nn.sigmoid(g[:, 384:512])
            c = fg * c + ig * gg
            h = og * jnp.tanh(c)

        qp = jnp.dot(q, wqp_ref[...], preferred_element_type=jnp.float32) + bqp_ref[...]
        pp = jnp.dot(h, wpp_ref[...], preferred_element_type=jnp.float32) + bpp_ref[...]
        ep = jnp.dot(ent, wef_ref[...], preferred_element_type=jnp.float32) + bef_ref[...]
        hh = jnp.maximum(jnp.dot(qp, wp1a_ref[...], preferred_element_type=jnp.float32)
                         + jnp.dot(pp, wp1b_ref[...], preferred_element_type=jnp.float32)
                         + jnp.dot(ep, wp1c_ref[...], preferred_element_type=jnp.float32)
                         + bp1_ref[...], 0.0)
        h2 = jnp.maximum(jnp.dot(hh, wp2_ref[...], preferred_element_type=jnp.float32)
                         + bp2_ref[...], 0.0)
        logits = jnp.dot(h2, wp3_ref[...], preferred_element_type=jnp.float32) + bp3_ref[...]

        vr = vr_ref[...]
        oh = (jax.lax.broadcasted_iota(jnp.int32, (1024, 64), 0) == vr).astype(jnp.float32)
        vl = jnp.dot(logits, oh, preferred_element_type=jnp.float32)
        mx = jnp.max(vl)
        ex = jnp.exp(vl - mx)
        probs_ref[...] = ex / jnp.sum(ex)
        vlog_ref[...] = vl


def kernel(question_emb, current_entity_emb, path_entities, path_relations,
           neighbor_entities, valid_relations,
           Wq1, bq1, Wq2, bq2, Wep, bep, W_ih, W_hh, b_ih, b_hh,
           Wg1, as1, ad1, bg1, Wg2, as2, ad2, bg2,
           Wqp, bqp, Wpp, bpp, Wef, bef, Wp1, bp1, Wp2, bp2, Wp3, bp3):
    f32 = jnp.float32
    bf16 = jnp.bfloat16
    M = neighbor_entities.shape[0]
    R1 = 4096
    T1 = (M + R1 - 1) // R1
    R2 = 8192
    T2 = (T1 * R1 + R2 - 1) // R2
    NREL = Wp3.shape[1]
    NRELP = ((NREL + 127) // 128) * 128
    NV = valid_relations.shape[0]

    idx = jnp.arange(512)
    hcol = idx // 128
    asp = jnp.zeros((512, 128), f32).at[idx, hcol].set(as1.reshape(-1)).astype(bf16)
    adp = jnp.zeros((512, 128), f32).at[idx, hcol].set(ad1.reshape(-1)).astype(bf16)
    Ef = (hcol[None, :] == jnp.arange(128)[:, None]).astype(f32)
    Eb = Ef.astype(bf16)
    Os2 = jnp.broadcast_to(as2.reshape(128, 1), (128, 128)).astype(bf16)

    x0r = current_entity_emb.reshape(1, 128).astype(bf16)
    row = lambda v: v.reshape(1, -1)

    full = lambda shp: pl.BlockSpec(shp, lambda i: tuple(0 for _ in shp))
    p1 = pl.pallas_call(
        functools.partial(_pass1_kernel, M, R1, T1),
        grid=(T1,),
        in_specs=[
            pl.BlockSpec((R1, 128), lambda i: (i, 0)),
            full((1, 128)),
            full((128, 512)),
            full((512, 128)),
            full((512, 128)),
            full((128, 512)),
            full((128, 512)),
            full((1, 512)),
            full((512, 128)),
            full((128, 128)),
            full((1, 128)),
            full((1, 128)),
        ],
        out_specs=[
            pl.BlockSpec((R1, 128), lambda i: (i, 0)),
            full((1, 128)),
            full((1, 128)),
        ],
        out_shape=[
            jax.ShapeDtypeStruct((T1 * R1, 128), bf16),
            jax.ShapeDtypeStruct((1, 128), f32),
            jax.ShapeDtypeStruct((1, 128), f32),
        ],
        scratch_shapes=[
            pltpu.VMEM((1, 128), f32),
            pltpu.VMEM((1, 128), f32),
            pltpu.VMEM((1, 512), f32),
            pltpu.VMEM((1, 128), f32),
        ],
    )
    xw2_buf, xw2_0, scal = p1(
        neighbor_entities, x0r, Wg1.astype(bf16), asp, adp,
        Eb, Ef, row(bg1).astype(bf16), Wg2.astype(bf16), Os2, as2, ad2)

    wp3p = jnp.zeros((Wp3.shape[0], NRELP), f32).at[:, :NREL].set(Wp3)
    bp3p = jnp.zeros((1, NRELP), f32).at[0, :NREL].set(bp3)

    p2 = pl.pallas_call(
        functools.partial(_pass2_kernel, M, R2, T2),
        grid=(T2,),
        in_specs=[
            pl.BlockSpec((R2, 128), lambda i: (i, 0)),
            full((1, 128)),
            full((1, 128)),
            full((128, 128)),
            full((1, 128)),
            full((1, 128)),
            full((128, 128)), full((1, 128)),
            full((128, 128)), full((1, 128)),
            full((5, 128)),
            full((5, 128)),
            full((128, 128)), full((1, 128)),
            full((128, 512)),
            full((128, 512)),
            full((1, 512)),
            full((128, 512)),
            full((1, 512)),
            full((128, 128)), full((1, 128)),
            full((128, 128)), full((1, 128)),
            full((128, 128)), full((1, 128)),
            full((128, 128)), full((128, 128)), full((128, 128)),
            full((1, 128)),
            full((128, 64)), full((1, 64)),
            full((64, NRELP)), full((1, NRELP)),
            full((1, NV)),
        ],
        out_specs=[full((1, NV)), full((1, NV))],
        out_shape=[
            jax.ShapeDtypeStruct((1, NV), f32),
            jax.ShapeDtypeStruct((1, NV), f32),
        ],
        scratch_shapes=[
            pltpu.VMEM((1, 128), f32),
            pltpu.VMEM((1, 128), f32),
        ],
    )
    probs, vlog = p2(
        xw2_buf, xw2_0, scal, Os2, row(bg2),
        row(question_emb), Wq1, row(bq1), Wq2, row(bq2),
        path_entities[:-1], path_relations, Wep, row(bep),
        W_ih[:, :128].T, W_ih[:, 128:].T, row(b_ih), W_hh.T, row(b_hh),
        Wqp, row(bqp), Wpp, row(bpp), Wef, row(bef),
        Wp1[0:128], Wp1[128:256], Wp1[256:384], row(bp1),
        Wp2, row(bp2), wp3p, bp3p, valid_relations.reshape(1, -1),
    )
    return probs.reshape(-1), vlog.reshape(-1)


# speedup vs baseline: 1.5805x; 1.2984x over previous
import functools

import jax
import jax.numpy as jnp
from jax.experimental import pallas as pl
from jax.experimental.pallas import tpu as pltpu

_NEG = -1e30


def _lrelu(x):
    return jnp.where(x > 0, x, 0.2 * x)


def _pass1_kernel(M, R, T,
                  x_ref, x0_ref, wg1_ref, wg1f_ref, asp_ref, adp_ref, ef_ref,
                  bg1_ref, wg2_ref, os2_ref, as2_ref, ad2_ref,
                  xw2_ref, xw20_ref, scal_ref,
                  m1_ref, s1_ref, accx_ref, mals2_ref, was_ref, wad_ref):
    i = pl.program_id(0)
    bf16 = jnp.bfloat16

    @pl.when(i == 0)
    def _init():
        m1_ref[...] = jnp.full((1, 128), _NEG, jnp.float32)
        s1_ref[...] = jnp.zeros((1, 128), jnp.float32)
        accx_ref[...] = jnp.zeros((128, 128), jnp.float32)
        mals2_ref[...] = jnp.full((1, 128), _NEG, jnp.float32)
        was_ref[...] = jnp.dot(wg1_ref[...], asp_ref[...],
                               preferred_element_type=jnp.float32).astype(bf16)
        wad_ref[...] = jnp.dot(wg1_ref[...], adp_ref[...],
                               preferred_element_type=jnp.float32).astype(bf16)

    base = i * R
    rid = jax.lax.broadcasted_iota(jnp.int32, (R, 128), 0)
    valid = (base + rid) < M

    Xt = jnp.where(valid, x_ref[...], 0.0).astype(bf16)
    XW = jnp.dot(Xt, wg1_ref[...], preferred_element_type=jnp.float32)
    ALS = jnp.dot(Xt, was_ref[...], preferred_element_type=jnp.float32)
    ALD = jnp.dot(Xt, wad_ref[...], preferred_element_type=jnp.float32)

    xw0 = jnp.dot(x0_ref[...], wg1_ref[...], preferred_element_type=jnp.float32)
    als0 = jnp.dot(x0_ref[...], was_ref[...], preferred_element_type=jnp.float32)
    ald0 = jnp.dot(x0_ref[...], wad_ref[...], preferred_element_type=jnp.float32)

    e0j = _lrelu(als0 + ALD)
    ejj = _lrelu(ALS + ALD)
    aj = jax.nn.sigmoid(ejj - e0j)
    Aje = jnp.concatenate(
        [jnp.broadcast_to(aj[:, h:h + 1], (R, 128)) for h in range(4)], axis=1)
    X1 = xw0 + Aje * (XW - xw0)
    X1 = jnp.maximum(X1 + bg1_ref[...], 0.0).astype(bf16)

    ej0 = jnp.where(valid, _lrelu(ALS + ald0), _NEG)
    tm = jnp.max(ej0, axis=0, keepdims=True)
    mold = m1_ref[...]
    mnew = jnp.maximum(mold, tm)
    scale = jnp.exp(mold - mnew)
    w = jnp.exp(ej0 - mnew)
    s1_ref[...] = s1_ref[...] * scale + jnp.sum(w, axis=0, keepdims=True)
    Cx = jax.lax.dot_general(Xt, w.astype(bf16), (((0,), (0,)), ((), ())),
                             preferred_element_type=jnp.float32)
    accx_ref[...] = accx_ref[...] * scale + Cx
    m1_ref[...] = mnew

    XW2 = jnp.dot(X1, wg2_ref[...], preferred_element_type=jnp.float32)
    XW2 = jnp.where(valid, XW2, 0.0)
    XW2b = XW2.astype(bf16)
    xw2_ref[...] = XW2b
    ALS2 = jnp.dot(XW2b, os2_ref[...], preferred_element_type=jnp.float32)
    ALS2 = jnp.where(valid, ALS2, _NEG)
    mals2_ref[...] = jnp.maximum(mals2_ref[...], jnp.max(ALS2, axis=0, keepdims=True))

    @pl.when(i == T - 1)
    def _epilogue():
        e00 = _lrelu(als0 + ald0)
        mo = m1_ref[...]
        mf = jnp.maximum(mo, e00)
        sc_o = jnp.exp(mo - mf)
        sc_s = jnp.exp(e00 - mf)
        s = s1_ref[...] * sc_o + sc_s
        Cmat = jax.lax.dot_general(accx_ref[...], wg1f_ref[...],
                                   (((0,), (0,)), ((), ())),
                                   preferred_element_type=jnp.float32)
        Crow = jnp.concatenate(
            [Cmat[h:h + 1, 128 * h:128 * (h + 1)] for h in range(4)], axis=1)
        accf = (Crow * jnp.dot(sc_o, ef_ref[...], preferred_element_type=jnp.float32)
                + jnp.dot(sc_s, ef_ref[...], preferred_element_type=jnp.float32) * xw0)
        sE = jnp.dot(s, ef_ref[...], preferred_element_type=jnp.float32)
        x1_0 = jnp.maximum(accf / sE + bg1_ref[...], 0.0)
        xw2_0 = jnp.dot(x1_0.astype(bf16), wg2_ref[...],
                        preferred_element_type=jnp.float32)
        xw20_ref[...] = xw2_0
        als2_0 = jnp.sum(xw2_0 * as2_ref[...])
        ald2_0 = jnp.sum(xw2_0 * ad2_ref[...])
        gmax = jnp.maximum(jnp.max(mals2_ref[...]), als2_0)
        lane = jax.lax.broadcasted_iota(jnp.int32, (1, 128), 1)
        scal_ref[...] = (jnp.where(lane == 0, als2_0, 0.0)
                         + jnp.where(lane == 1, ald2_0, 0.0)
                         + jnp.where(lane == 2, gmax, 0.0))


def _pass2_kernel(M, R, T,
                  xw2_ref, xw20_ref, scal_ref, os2_ref, bg2_ref,
                  qe_ref, wq1_ref, bq1_ref, wq2_ref, bq2_ref,
                  pe_ref, pr_ref, wep_ref, bep_ref,
                  wihe_ref, wihr_ref, bih_ref, whh_ref, bhh_ref,
                  wqp_ref, bqp_ref, wpp_ref, bpp_ref, wef_ref, bef_ref,
                  wp1a_ref, wp1b_ref, wp1c_ref, bp1_ref,
                  wp2_ref, bp2_ref, wp3_ref, bp3_ref, vr_ref,
                  probs_ref, vlog_ref,
                  s2_ref, acc2_ref):
    i = pl.program_id(0)

    @pl.when(i == 0)
    def _init():
        s2_ref[...] = jnp.zeros((1, 128), jnp.float32)
        acc2_ref[...] = jnp.zeros((1, 128), jnp.float32)

    lane = jax.lax.broadcasted_iota(jnp.int32, (1, 128), 1)
    scal = scal_ref[...]
    als2_0 = jnp.sum(jnp.where(lane == 0, scal, 0.0))
    ald2_0 = jnp.sum(jnp.where(lane == 1, scal, 0.0))
    gmax = jnp.sum(jnp.where(lane == 2, scal, 0.0))
    m2 = _lrelu(gmax + ald2_0)

    XW2b = xw2_ref[...]
    XW2 = XW2b.astype(jnp.float32)
    rid = jax.lax.broadcasted_iota(jnp.int32, (R, 128), 0)
    valid = (i * R + rid) < M
    XW2 = jnp.where(valid, XW2, 0.0)
    ALS2 = jnp.dot(XW2b, os2_ref[...], preferred_element_type=jnp.float32)
    e2 = jnp.where(valid, _lrelu(ALS2 + ald2_0), _NEG)
    w = jnp.exp(e2 - m2)
    acc2_ref[...] = acc2_ref[...] + jnp.sum(w * XW2, axis=0, keepdims=True)
    s2_ref[...] = s2_ref[...] + jnp.sum(w, axis=0, keepdims=True)

    @pl.when(i == T - 1)
    def _epilogue():
        e00 = _lrelu(als2_0 + ald2_0)
        w00 = jnp.exp(e00 - m2)
        acc = acc2_ref[...] + w00 * xw20_ref[...]
        s = s2_ref[...] + w00
        ent = jnp.maximum(acc / s + bg2_ref[...], 0.0)

        q = jnp.maximum(jnp.dot(qe_ref[...], wq1_ref[...],
                                preferred_element_type=jnp.float32) + bq1_ref[...], 0.0)
        q = jnp.dot(q, wq2_ref[...], preferred_element_type=jnp.float32) + bq2_ref[...]

        ents = jnp.dot(pe_ref[...], wep_ref[...],
                       preferred_element_type=jnp.float32) + bep_ref[...]
        prel = pr_ref[...]
        h = jnp.zeros((1, 128), jnp.float32)
        c = jnp.zeros((1, 128), jnp.float32)
        for t in range(5):
            g = (jnp.dot(ents[t:t + 1, :], wihe_ref[...], preferred_element_type=jnp.float32)
                 + jnp.dot(prel[t:t + 1, :], wihr_ref[...], preferred_element_type=jnp.float32)
                 + bih_ref[...]
                 + jnp.dot(h, whh_ref[...], preferred_element_type=jnp.float32)
                 + bhh_ref[...])
            ig = jax.nn.sigmoid(g[:, 0:128])
            fg = jax.nn.sigmoid(g[:, 128:256])
            gg = jnp.tanh(g[:, 256:384])
            og = jax.nn.sigmoid(g[:, 384:512])
            c = fg * c + ig * gg
            h = og * jnp.tanh(c)

        qp = jnp.dot(q, wqp_ref[...], preferred_element_type=jnp.float32) + bqp_ref[...]
        pp = jnp.dot(h, wpp_ref[...], preferred_element_type=jnp.float32) + bpp_ref[...]
        ep = jnp.dot(ent, wef_ref[...], preferred_element_type=jnp.float32) + bef_ref[...]
        hh = jnp.maximum(jnp.dot(qp, wp1a_ref[...], preferred_element_type=jnp.float32)
                         + jnp.dot(pp, wp1b_ref[...], preferred_element_type=jnp.float32)
                         + jnp.dot(ep, wp1c_ref[...], preferred_element_type=jnp.float32)
                         + bp1_ref[...], 0.0)
        h2 = jnp.maximum(jnp.dot(hh, wp2_ref[...], preferred_element_type=jnp.float32)
                         + bp2_ref[...], 0.0)
        logits = jnp.dot(h2, wp3_ref[...], preferred_element_type=jnp.float32) + bp3_ref[...]

        vr = vr_ref[...]
        oh = (jax.lax.broadcasted_iota(jnp.int32, (1024, 64), 0) == vr).astype(jnp.float32)
        vl = jnp.dot(logits, oh, preferred_element_type=jnp.float32)
        mx = jnp.max(vl)
        ex = jnp.exp(vl - mx)
        probs_ref[...] = ex / jnp.sum(ex)
        vlog_ref[...] = vl


def kernel(question_emb, current_entity_emb, path_entities, path_relations,
           neighbor_entities, valid_relations,
           Wq1, bq1, Wq2, bq2, Wep, bep, W_ih, W_hh, b_ih, b_hh,
           Wg1, as1, ad1, bg1, Wg2, as2, ad2, bg2,
           Wqp, bqp, Wpp, bpp, Wef, bef, Wp1, bp1, Wp2, bp2, Wp3, bp3):
    f32 = jnp.float32
    bf16 = jnp.bfloat16
    M = neighbor_entities.shape[0]
    R1 = 4096
    T1 = (M + R1 - 1) // R1
    R2 = 8192
    T2 = (T1 * R1 + R2 - 1) // R2
    NREL = Wp3.shape[1]
    NRELP = ((NREL + 127) // 128) * 128
    NV = valid_relations.shape[0]

    idx = jnp.arange(512)
    hcol = idx // 128
    asp = jnp.zeros((512, 128), f32).at[idx, hcol].set(as1.reshape(-1)).astype(bf16)
    adp = jnp.zeros((512, 128), f32).at[idx, hcol].set(ad1.reshape(-1)).astype(bf16)
    Ef = (hcol[None, :] == jnp.arange(128)[:, None]).astype(f32)
    Eb = Ef.astype(bf16)
    Os2 = jnp.broadcast_to(as2.reshape(128, 1), (128, 128)).astype(bf16)

    x0r = current_entity_emb.reshape(1, 128).astype(bf16)
    row = lambda v: v.reshape(1, -1)

    full = lambda shp: pl.BlockSpec(shp, lambda i: tuple(0 for _ in shp))
    p1 = pl.pallas_call(
        functools.partial(_pass1_kernel, M, R1, T1),
        grid=(T1,),
        in_specs=[
            pl.BlockSpec((R1, 128), lambda i: (i, 0)),
            full((1, 128)),
            full((128, 512)),
            full((128, 512)),
            full((512, 128)),
            full((512, 128)),
            full((128, 512)),
            full((1, 512)),
            full((512, 128)),
            full((128, 128)),
            full((1, 128)),
            full((1, 128)),
        ],
        out_specs=[
            pl.BlockSpec((R1, 128), lambda i: (i, 0)),
            full((1, 128)),
            full((1, 128)),
        ],
        out_shape=[
            jax.ShapeDtypeStruct((T1 * R1, 128), bf16),
            jax.ShapeDtypeStruct((1, 128), f32),
            jax.ShapeDtypeStruct((1, 128), f32),
        ],
        scratch_shapes=[
            pltpu.VMEM((1, 128), f32),
            pltpu.VMEM((1, 128), f32),
            pltpu.VMEM((128, 128), f32),
            pltpu.VMEM((1, 128), f32),
            pltpu.VMEM((128, 128), bf16),
            pltpu.VMEM((128, 128), bf16),
        ],
    )
    xw2_buf, xw2_0, scal = p1(
        neighbor_entities, x0r, Wg1.astype(bf16), Wg1, asp, adp,
        Ef, row(bg1).astype(bf16), Wg2.astype(bf16), Os2, as2, ad2)

    wp3p = jnp.zeros((Wp3.shape[0], NRELP), f32).at[:, :NREL].set(Wp3)
    bp3p = jnp.zeros((1, NRELP), f32).at[0, :NREL].set(bp3)

    p2 = pl.pallas_call(
        functools.partial(_pass2_kernel, M, R2, T2),
        grid=(T2,),
        in_specs=[
            pl.BlockSpec((R2, 128), lambda i: (i, 0)),
            full((1, 128)),
            full((1, 128)),
            full((128, 128)),
            full((1, 128)),
            full((1, 128)),
            full((128, 128)), full((1, 128)),
            full((128, 128)), full((1, 128)),
            full((5, 128)),
            full((5, 128)),
            full((128, 128)), full((1, 128)),
            full((128, 512)),
            full((128, 512)),
            full((1, 512)),
            full((128, 512)),
            full((1, 512)),
            full((128, 128)), full((1, 128)),
            full((128, 128)), full((1, 128)),
            full((128, 128)), full((1, 128)),
            full((128, 128)), full((128, 128)), full((128, 128)),
            full((1, 128)),
            full((128, 64)), full((1, 64)),
            full((64, NRELP)), full((1, NRELP)),
            full((1, NV)),
        ],
        out_specs=[full((1, NV)), full((1, NV))],
        out_shape=[
            jax.ShapeDtypeStruct((1, NV), f32),
            jax.ShapeDtypeStruct((1, NV), f32),
        ],
        scratch_shapes=[
            pltpu.VMEM((1, 128), f32),
            pltpu.VMEM((1, 128), f32),
        ],
    )
    probs, vlog = p2(
        xw2_buf, xw2_0, scal, Os2, row(bg2),
        row(question_emb), Wq1, row(bq1), Wq2, row(bq2),
        path_entities[:-1], path_relations, Wep, row(bep),
        W_ih[:, :128].T, W_ih[:, 128:].T, row(b_ih), W_hh.T, row(b_hh),
        Wqp, row(bqp), Wpp, row(bpp), Wef, row(bef),
        Wp1[0:128], Wp1[128:256], Wp1[256:384], row(bp1),
        Wp2, row(bp2), wp3p, bp3p, valid_relations.reshape(1, -1),
    )
    return probs.reshape(-1), vlog.reshape(-1)


# speedup vs baseline: 1.6217x; 1.0260x over previous
import functools

import jax
import jax.numpy as jnp
from jax.experimental import pallas as pl
from jax.experimental.pallas import tpu as pltpu

_NEG = -1e30


def _lrelu(x):
    return jnp.where(x > 0, x, 0.2 * x)


def _pass1_kernel(M, R, T,
                  x_ref, x0_ref, wg1_ref, wg1f_ref, asp_ref, adp_ref, ef_ref,
                  bg1_ref, wg2_ref, os2_ref, as2_ref, ad2_ref,
                  xw2_ref, xw20_ref, scal_ref,
                  m1_ref, s1_ref, accx_ref, mals2_ref, was_ref, wad_ref):
    i = pl.program_id(0)
    bf16 = jnp.bfloat16

    @pl.when(i == 0)
    def _init():
        m1_ref[...] = jnp.full((1, 128), _NEG, jnp.float32)
        s1_ref[...] = jnp.zeros((1, 128), jnp.float32)
        accx_ref[...] = jnp.zeros((128, 128), jnp.float32)
        mals2_ref[...] = jnp.full((1, 128), _NEG, jnp.float32)
        was_ref[...] = jnp.dot(wg1_ref[...], asp_ref[...],
                               preferred_element_type=jnp.float32).astype(bf16)
        wad_ref[...] = jnp.dot(wg1_ref[...], adp_ref[...],
                               preferred_element_type=jnp.float32).astype(bf16)

    base = i * R
    rid = jax.lax.broadcasted_iota(jnp.int32, (R, 128), 0)
    valid = (base + rid) < M

    Xt = jnp.where(valid, x_ref[...], 0.0).astype(bf16)
    XW = jnp.dot(Xt, wg1_ref[...], preferred_element_type=jnp.float32)
    ALS = jnp.dot(Xt, was_ref[...], preferred_element_type=jnp.float32)
    ALD = jnp.dot(Xt, wad_ref[...], preferred_element_type=jnp.float32)

    xw0 = jnp.dot(x0_ref[...], wg1_ref[...], preferred_element_type=jnp.float32)
    als0 = jnp.dot(x0_ref[...], was_ref[...], preferred_element_type=jnp.float32)
    ald0 = jnp.dot(x0_ref[...], wad_ref[...], preferred_element_type=jnp.float32)

    e0j = _lrelu(als0 + ALD)
    ejj = _lrelu(ALS + ALD)
    aj = jax.nn.sigmoid(ejj - e0j).astype(bf16)
    Aje = jnp.concatenate(
        [jnp.broadcast_to(aj[:, h:h + 1], (R, 128)) for h in range(4)], axis=1)
    XWb = XW.astype(bf16)
    xw0b = xw0.astype(bf16)
    X1 = xw0b + Aje * (XWb - xw0b)
    X1 = jnp.maximum(X1 + bg1_ref[...], bf16(0))

    ej0 = jnp.where(valid, _lrelu(ALS + ald0), _NEG)
    tm = jnp.max(ej0, axis=0, keepdims=True)
    mold = m1_ref[...]
    mnew = jnp.maximum(mold, tm)
    scale = jnp.exp(mold - mnew)
    w = jnp.exp(ej0 - mnew)
    s1_ref[...] = s1_ref[...] * scale + jnp.sum(w, axis=0, keepdims=True)
    Cx = jax.lax.dot_general(Xt, w.astype(bf16), (((0,), (0,)), ((), ())),
                             preferred_element_type=jnp.float32)
    accx_ref[...] = accx_ref[...] * scale + Cx
    m1_ref[...] = mnew

    XW2 = jnp.dot(X1, wg2_ref[...], preferred_element_type=jnp.float32)
    XW2 = jnp.where(valid, XW2, 0.0)
    XW2b = XW2.astype(bf16)
    xw2_ref[...] = XW2b
    ALS2 = jnp.dot(XW2b, os2_ref[...], preferred_element_type=jnp.float32)
    ALS2 = jnp.where(valid, ALS2, _NEG)
    mals2_ref[...] = jnp.maximum(mals2_ref[...], jnp.max(ALS2, axis=0, keepdims=True))

    @pl.when(i == T - 1)
    def _epilogue():
        e00 = _lrelu(als0 + ald0)
        mo = m1_ref[...]
        mf = jnp.maximum(mo, e00)
        sc_o = jnp.exp(mo - mf)
        sc_s = jnp.exp(e00 - mf)
        s = s1_ref[...] * sc_o + sc_s
        Cmat = jax.lax.dot_general(accx_ref[...], wg1f_ref[...],
                                   (((0,), (0,)), ((), ())),
                                   preferred_element_type=jnp.float32)
        Crow = jnp.concatenate(
            [Cmat[h:h + 1, 128 * h:128 * (h + 1)] for h in range(4)], axis=1)
        accf = (Crow * jnp.dot(sc_o, ef_ref[...], preferred_element_type=jnp.float32)
                + jnp.dot(sc_s, ef_ref[...], preferred_element_type=jnp.float32) * xw0)
        sE = jnp.dot(s, ef_ref[...], preferred_element_type=jnp.float32)
        x1_0 = jnp.maximum(accf / sE + bg1_ref[...], 0.0)
        xw2_0 = jnp.dot(x1_0.astype(bf16), wg2_ref[...],
                        preferred_element_type=jnp.float32)
        xw20_ref[...] = xw2_0
        als2_0 = jnp.sum(xw2_0 * as2_ref[...])
        ald2_0 = jnp.sum(xw2_0 * ad2_ref[...])
        gmax = jnp.maximum(jnp.max(mals2_ref[...]), als2_0)
        lane = jax.lax.broadcasted_iota(jnp.int32, (1, 128), 1)
        scal_ref[...] = (jnp.where(lane == 0, als2_0, 0.0)
                         + jnp.where(lane == 1, ald2_0, 0.0)
                         + jnp.where(lane == 2, gmax, 0.0))


def _pass2_kernel(M, R, T,
                  xw2_ref, xw20_ref, scal_ref, os2_ref, bg2_ref,
                  qe_ref, wq1_ref, bq1_ref, wq2_ref, bq2_ref,
                  pe_ref, pr_ref, wep_ref, bep_ref,
                  wihe_ref, wihr_ref, bih_ref, whh_ref, bhh_ref,
                  wqp_ref, bqp_ref, wpp_ref, bpp_ref, wef_ref, bef_ref,
                  wp1a_ref, wp1b_ref, wp1c_ref, bp1_ref,
                  wp2_ref, bp2_ref, wp3_ref, bp3_ref, vr_ref,
                  probs_ref, vlog_ref,
                  s2_ref, acc2_ref):
    i = pl.program_id(0)

    @pl.when(i == 0)
    def _init():
        s2_ref[...] = jnp.zeros((1, 128), jnp.float32)
        acc2_ref[...] = jnp.zeros((1, 128), jnp.float32)

    lane = jax.lax.broadcasted_iota(jnp.int32, (1, 128), 1)
    scal = scal_ref[...]
    als2_0 = jnp.sum(jnp.where(lane == 0, scal, 0.0))
    ald2_0 = jnp.sum(jnp.where(lane == 1, scal, 0.0))
    gmax = jnp.sum(jnp.where(lane == 2, scal, 0.0))
    m2 = _lrelu(gmax + ald2_0)

    XW2b = xw2_ref[...]
    XW2 = XW2b.astype(jnp.float32)
    rid = jax.lax.broadcasted_iota(jnp.int32, (R, 128), 0)
    valid = (i * R + rid) < M
    XW2 = jnp.where(valid, XW2, 0.0)
    ALS2 = jnp.dot(XW2b, os2_ref[...], preferred_element_type=jnp.float32)
    e2 = jnp.where(valid, _lrelu(ALS2 + ald2_0), _NEG)
    w = jnp.exp(e2 - m2)
    acc2_ref[...] = acc2_ref[...] + jnp.sum(w * XW2, axis=0, keepdims=True)
    s2_ref[...] = s2_ref[...] + jnp.sum(w, axis=0, keepdims=True)

    @pl.when(i == T - 1)
    def _epilogue():
        e00 = _lrelu(als2_0 + ald2_0)
        w00 = jnp.exp(e00 - m2)
        acc = acc2_ref[...] + w00 * xw20_ref[...]
        s = s2_ref[...] + w00
        ent = jnp.maximum(acc / s + bg2_ref[...], 0.0)

        q = jnp.maximum(jnp.dot(qe_ref[...], wq1_ref[...],
                                preferred_element_type=jnp.float32) + bq1_ref[...], 0.0)
        q = jnp.dot(q, wq2_ref[...], preferred_element_type=jnp.float32) + bq2_ref[...]

        ents = jnp.dot(pe_ref[...], wep_ref[...],
                       preferred_element_type=jnp.float32) + bep_ref[...]
        prel = pr_ref[...]
        h = jnp.zeros((1, 128), jnp.float32)
        c = jnp.zeros((1, 128), jnp.float32)
        for t in range(5):
            g = (jnp.dot(ents[t:t + 1, :], wihe_ref[...], preferred_element_type=jnp.float32)
                 + jnp.dot(prel[t:t + 1, :], wihr_ref[...], preferred_element_type=jnp.float32)
                 + bih_ref[...]
                 + jnp.dot(h, whh_ref[...], preferred_element_type=jnp.float32)
                 + bhh_ref[...])
            ig = jax.nn.sigmoid(g[:, 0:128])
            fg = jax.nn.sigmoid(g[:, 128:256])
            gg = jnp.tanh(g[:, 256:384])
            og = jax.nn.sigmoid(g[:, 384:512])
            c = fg * c + ig * gg
            h = og * jnp.tanh(c)

        qp = jnp.dot(q, wqp_ref[...], preferred_element_type=jnp.float32) + bqp_ref[...]
        pp = jnp.dot(h, wpp_ref[...], preferred_element_type=jnp.float32) + bpp_ref[...]
        ep = jnp.dot(ent, wef_ref[...], preferred_element_type=jnp.float32) + bef_ref[...]
        hh = jnp.maximum(jnp.dot(qp, wp1a_ref[...], preferred_element_type=jnp.float32)
                         + jnp.dot(pp, wp1b_ref[...], preferred_element_type=jnp.float32)
                         + jnp.dot(ep, wp1c_ref[...], preferred_element_type=jnp.float32)
                         + bp1_ref[...], 0.0)
        h2 = jnp.maximum(jnp.dot(hh, wp2_ref[...], preferred_element_type=jnp.float32)
                         + bp2_ref[...], 0.0)
        logits = jnp.dot(h2, wp3_ref[...], preferred_element_type=jnp.float32) + bp3_ref[...]

        vr = vr_ref[...]
        oh = (jax.lax.broadcasted_iota(jnp.int32, (1024, 64), 0) == vr).astype(jnp.float32)
        vl = jnp.dot(logits, oh, preferred_element_type=jnp.float32)
        mx = jnp.max(vl)
        ex = jnp.exp(vl - mx)
        probs_ref[...] = ex / jnp.sum(ex)
        vlog_ref[...] = vl


def kernel(question_emb, current_entity_emb, path_entities, path_relations,
           neighbor_entities, valid_relations,
           Wq1, bq1, Wq2, bq2, Wep, bep, W_ih, W_hh, b_ih, b_hh,
           Wg1, as1, ad1, bg1, Wg2, as2, ad2, bg2,
           Wqp, bqp, Wpp, bpp, Wef, bef, Wp1, bp1, Wp2, bp2, Wp3, bp3):
    f32 = jnp.float32
    bf16 = jnp.bfloat16
    M = neighbor_entities.shape[0]
    R1 = 4096
    T1 = (M + R1 - 1) // R1
    R2 = 8192
    T2 = (T1 * R1 + R2 - 1) // R2
    NREL = Wp3.shape[1]
    NRELP = ((NREL + 127) // 128) * 128
    NV = valid_relations.shape[0]

    idx = jnp.arange(512)
    hcol = idx // 128
    asp = jnp.zeros((512, 128), f32).at[idx, hcol].set(as1.reshape(-1)).astype(bf16)
    adp = jnp.zeros((512, 128), f32).at[idx, hcol].set(ad1.reshape(-1)).astype(bf16)
    Ef = (hcol[None, :] == jnp.arange(128)[:, None]).astype(f32)
    Eb = Ef.astype(bf16)
    Os2 = jnp.broadcast_to(as2.reshape(128, 1), (128, 128)).astype(bf16)

    x0r = current_entity_emb.reshape(1, 128).astype(bf16)
    row = lambda v: v.reshape(1, -1)

    full = lambda shp: pl.BlockSpec(shp, lambda i: tuple(0 for _ in shp))
    p1 = pl.pallas_call(
        functools.partial(_pass1_kernel, M, R1, T1),
        grid=(T1,),
        in_specs=[
            pl.BlockSpec((R1, 128), lambda i: (i, 0)),
            full((1, 128)),
            full((128, 512)),
            full((128, 512)),
            full((512, 128)),
            full((512, 128)),
            full((128, 512)),
            full((1, 512)),
            full((512, 128)),
            full((128, 128)),
            full((1, 128)),
            full((1, 128)),
        ],
        out_specs=[
            pl.BlockSpec((R1, 128), lambda i: (i, 0)),
            full((1, 128)),
            full((1, 128)),
        ],
        out_shape=[
            jax.ShapeDtypeStruct((T1 * R1, 128), bf16),
            jax.ShapeDtypeStruct((1, 128), f32),
            jax.ShapeDtypeStruct((1, 128), f32),
        ],
        scratch_shapes=[
            pltpu.VMEM((1, 128), f32),
            pltpu.VMEM((1, 128), f32),
            pltpu.VMEM((128, 128), f32),
            pltpu.VMEM((1, 128), f32),
            pltpu.VMEM((128, 128), bf16),
            pltpu.VMEM((128, 128), bf16),
        ],
    )
    xw2_buf, xw2_0, scal = p1(
        neighbor_entities, x0r, Wg1.astype(bf16), Wg1, asp, adp,
        Ef, row(bg1).astype(bf16), Wg2.astype(bf16), Os2, as2, ad2)

    wp3p = jnp.zeros((Wp3.shape[0], NRELP), f32).at[:, :NREL].set(Wp3)
    bp3p = jnp.zeros((1, NRELP), f32).at[0, :NREL].set(bp3)

    p2 = pl.pallas_call(
        functools.partial(_pass2_kernel, M, R2, T2),
        grid=(T2,),
        in_specs=[
            pl.BlockSpec((R2, 128), lambda i: (i, 0)),
            full((1, 128)),
            full((1, 128)),
            full((128, 128)),
            full((1, 128)),
            full((1, 128)),
            full((128, 128)), full((1, 128)),
            full((128, 128)), full((1, 128)),
            full((5, 128)),
            full((5, 128)),
            full((128, 128)), full((1, 128)),
            full((128, 512)),
            full((128, 512)),
            full((1, 512)),
            full((128, 512)),
            full((1, 512)),
            full((128, 128)), full((1, 128)),
            full((128, 128)), full((1, 128)),
            full((128, 128)), full((1, 128)),
            full((128, 128)), full((128, 128)), full((128, 128)),
            full((1, 128)),
            full((128, 64)), full((1, 64)),
            full((64, NRELP)), full((1, NRELP)),
            full((1, NV)),
        ],
        out_specs=[full((1, NV)), full((1, NV))],
        out_shape=[
            jax.ShapeDtypeStruct((1, NV), f32),
            jax.ShapeDtypeStruct((1, NV), f32),
        ],
        scratch_shapes=[
            pltpu.VMEM((1, 128), f32),
            pltpu.VMEM((1, 128), f32),
        ],
    )
    probs, vlog = p2(
        xw2_buf, xw2_0, scal, Os2, row(bg2),
        row(question_emb), Wq1, row(bq1), Wq2, row(bq2),
        path_entities[:-1], path_relations, Wep, row(bep),
        W_ih[:, :128].T, W_ih[:, 128:].T, row(b_ih), W_hh.T, row(b_hh),
        Wqp, row(bqp), Wpp, row(bpp), Wef, row(bef),
        Wp1[0:128], Wp1[128:256], Wp1[256:384], row(bp1),
        Wp2, row(bp2), wp3p, bp3p, valid_relations.reshape(1, -1),
    )
    return probs.reshape(-1), vlog.reshape(-1)
